# baseline (speedup 1.0000x reference)
import functools

import jax
import jax.numpy as jnp
from jax import lax
from jax.experimental import pallas as pl
from jax.experimental.pallas import tpu as pltpu
from jax.experimental.pallas import tpu_sc as plsc

N_CLUSTERS = 1024
EMBED_DIM = 256
DECAY = 0.99
EPS = 1e-05
BATCH = 16384

_BB = 2048


def _assign_body(xn_ref, xn2_ref, c_ref, c2_ref, idx_ref, slab_ref):
    xn = xn_ref[...]
    c = c_ref[...]
    mm = lax.dot_general(xn, c, (((1,), (1,)), ((), ())))
    d = (xn2_ref[...] + c2_ref[...]) - 2.0 * mm
    m = jnp.min(d, axis=1, keepdims=True)
    ii = lax.broadcasted_iota(jnp.int32, d.shape, 1)
    idx = jnp.min(jnp.where(d == m, ii, N_CLUSTERS), axis=1, keepdims=True)
    idx_ref[...] = idx
    slab_ref[...] = idx.reshape(1, 1, _BB)


def _assign(xn, xn2, c2, centroids):
    return pl.pallas_call(
        _assign_body,
        grid=(BATCH // _BB,),
        in_specs=[
            pl.BlockSpec((_BB, EMBED_DIM), lambda i: (i, 0)),
            pl.BlockSpec((_BB, 1), lambda i: (i, 0)),
            pl.BlockSpec((N_CLUSTERS, EMBED_DIM), lambda i: (0, 0)),
            pl.BlockSpec((1, N_CLUSTERS), lambda i: (0, 0)),
        ],
        out_specs=[
            pl.BlockSpec((_BB, 1), lambda i: (i, 0)),
            pl.BlockSpec((1, 1, _BB), lambda i: (i, 0, 0)),
        ],
        out_shape=[
            jax.ShapeDtypeStruct((BATCH, 1), jnp.int32),
            jax.ShapeDtypeStruct((BATCH // _BB, 1, _BB), jnp.int32),
        ],
    )(xn, xn2, centroids, c2)


_NC, _NS, _L = 2, 16, 16
_NSL = 4
_NG = 8
_CS = EMBED_DIM // _NSL
_RG = BATCH // _NG
_CHR = 256
_PAD = _CS + 1

_sc_mesh = plsc.VectorSubcoreMesh(core_axis_name="c", subcore_axis_name="s")


@functools.partial(
    pl.kernel,
    mesh=_sc_mesh,
    out_type=jax.ShapeDtypeStruct((BATCH, EMBED_DIM), jnp.float32),
    scratch_types=[
        pltpu.VMEM((_RG,), jnp.int32),
        pltpu.VMEM((N_CLUSTERS, _PAD), jnp.float32),
        pltpu.VMEM((_CHR, _PAD), jnp.float32),
        pltpu.VMEM((_CHR, _PAD), jnp.float32),
        pltpu.SemaphoreType.DMA,
        pltpu.SemaphoreType.DMA,
    ],
    compiler_params=pltpu.CompilerParams(use_tc_tiling_on_sc=False,
                                        needs_layout_passes=False),
)
def _sc_gather(idx8_hbm, cent_hbm, quant_hbm, idxbuf, tbl, ob0, ob1,
               s0, s1):
    cid = lax.axis_index("c")
    sid = lax.axis_index("s")
    sl = sid % _NSL
    rg = cid * (_NG // _NC) + sid // _NSL
    base = rg * _RG
    col0 = sl * _CS
    pltpu.sync_copy(idx8_hbm.at[rg], idxbuf)
    pltpu.sync_copy(cent_hbm.at[:, pl.ds(col0, _CS)], tbl.at[:, pl.ds(0, _CS)])
    lane = lax.iota(jnp.int32, _L)
    obufs = (ob0, ob1)
    sems = (s0, s1)
    nchunk = _RG // _CHR

    def _chunk(ci, carry):
        for p in range(2):
            @pl.when(lax.rem(ci, 2) == p)
            def _():
                ob = obufs[p]
                sem = sems[p]

                @pl.when(ci >= 2)
                def _():
                    pltpu.make_async_copy(
                        ob.at[:, pl.ds(0, _CS)],
                        quant_hbm.at[pl.ds(base, _CHR), pl.ds(col0, _CS)],
                        sem).wait()

                def _group(gi, c2):
                    idx16 = idxbuf[pl.ds(ci * _CHR + gi * _L, _L)]
                    row16 = gi * _L + lane
                    for c0 in range(0, _CS, 16):
                        ccs = [jnp.full((_L,), c0 + k, jnp.int32)
                               for k in range(16)]
                        vs = [plsc.load_gather(tbl, [idx16, cc])
                              for cc in ccs]
                        for cc, v in zip(ccs, vs):
                            plsc.store_scatter(ob, [row16, cc], v)
                    return c2

                lax.fori_loop(0, _CHR // _L, _group, 0, unroll=2)
                pltpu.async_copy(
                    ob.at[:, pl.ds(0, _CS)],
                    quant_hbm.at[pl.ds(base + ci * _CHR, _CHR),
                                 pl.ds(col0, _CS)], sem)
        return carry

    lax.fori_loop(0, nchunk, _chunk, 0)
    for p in range(2):
        pltpu.make_async_copy(
            obufs[p].at[:, pl.ds(0, _CS)],
            quant_hbm.at[pl.ds(base, _CHR), pl.ds(col0, _CS)],
            sems[p]).wait()


def _update_body(idx_ref, x_ref, ema_sz_ref, ema_w_ref, out_ref,
                 dw_acc, cnt_acc):
    i = pl.program_id(0)
    ii = lax.broadcasted_iota(jnp.int32, (_BB, N_CLUSTERS), 1)
    oh = (ii == idx_ref[...]).astype(jnp.float32)
    ones_col = jnp.ones((_BB, 1), jnp.float32)
    dw_p = lax.dot_general(oh, x_ref[...], (((0,), (0,)), ((), ())))
    cnt_p = lax.dot_general(oh, ones_col, (((0,), (0,)), ((), ())))

    @pl.when(i == 0)
    def _():
        dw_acc[...] = dw_p
        cnt_acc[...] = cnt_p

    @pl.when(i > 0)
    def _():
        dw_acc[...] += dw_p
        cnt_acc[...] += cnt_p

    @pl.when(i == pl.num_programs(0) - 1)
    def _():
        ns = ema_sz_ref[...] * DECAY + (1.0 - DECAY) * cnt_acc[...]
        n = jnp.sum(ns)
        ns2 = (ns + EPS) / (n + N_CLUSTERS * EPS) * n
        new_ema_w = ema_w_ref[...] * DECAY + (1.0 - DECAY) * dw_acc[...]
        out_ref[...] = new_ema_w / ns2


def _update(idx2, x, ema_sz, ema_w):
    return pl.pallas_call(
        _update_body,
        grid=(BATCH // _BB,),
        in_specs=[
            pl.BlockSpec((_BB, 1), lambda i: (i, 0)),
            pl.BlockSpec((_BB, EMBED_DIM), lambda i: (i, 0)),
            pl.BlockSpec((N_CLUSTERS, 1), lambda i: (0, 0)),
            pl.BlockSpec((N_CLUSTERS, EMBED_DIM), lambda i: (0, 0)),
        ],
        out_specs=pl.BlockSpec((N_CLUSTERS, EMBED_DIM), lambda i: (0, 0)),
        out_shape=jax.ShapeDtypeStruct((N_CLUSTERS, EMBED_DIM), jnp.float32),
        scratch_shapes=[
            pltpu.VMEM((N_CLUSTERS, EMBED_DIM), jnp.float32),
            pltpu.VMEM((N_CLUSTERS, 1), jnp.float32),
        ],
    )(idx2, x, ema_sz, ema_w)


def kernel(X, centroids, ema_cluster_size, ema_w):
    norm = jnp.linalg.norm(X, ord=2, axis=1, keepdims=True)
    Xn = X / jnp.clip(norm, 1e-12, None)
    xn2 = jnp.sum(Xn ** 2, axis=1, keepdims=True)
    c2 = jnp.sum(centroids ** 2, axis=1)[None, :]

    idx2, slab = _assign(Xn, xn2, c2, centroids)
    idx8 = slab.reshape(_NG, _RG)

    new_centroids = _update(idx2, X, ema_cluster_size.reshape(N_CLUSTERS, 1),
                            ema_w)
    quant = _sc_gather(idx8, centroids)
    return quant, idx2, new_centroids


# speedup vs baseline: 1.0784x; 1.0784x over previous
import functools

import jax
import jax.numpy as jnp
from jax import lax
from jax.experimental import pallas as pl
from jax.experimental.pallas import tpu as pltpu
from jax.experimental.pallas import tpu_sc as plsc

N_CLUSTERS = 1024
EMBED_DIM = 256
DECAY = 0.99
EPS = 1e-05
BATCH = 16384

_BB = 2048


def _assign_body(xn_ref, xn2_ref, c_ref, c2_ref, idx_ref):
    xn = xn_ref[...]
    c = c_ref[...]
    mm = lax.dot_general(xn, c, (((1,), (1,)), ((), ())))
    d = (xn2_ref[...] + c2_ref[...]) - 2.0 * mm
    m = jnp.min(d, axis=1, keepdims=True)
    ii = lax.broadcasted_iota(jnp.int32, d.shape, 1)
    idx_ref[...] = jnp.min(jnp.where(d == m, ii, N_CLUSTERS), axis=1,
                           keepdims=True)


def _assign(xn, xn2, c2, centroids):
    return pl.pallas_call(
        _assign_body,
        grid=(BATCH // _BB,),
        in_specs=[
            pl.BlockSpec((_BB, EMBED_DIM), lambda i: (i, 0)),
            pl.BlockSpec((_BB, 1), lambda i: (i, 0)),
            pl.BlockSpec((N_CLUSTERS, EMBED_DIM), lambda i: (0, 0)),
            pl.BlockSpec((1, N_CLUSTERS), lambda i: (0, 0)),
        ],
        out_specs=pl.BlockSpec((_BB, 1), lambda i: (i, 0)),
        out_shape=jax.ShapeDtypeStruct((BATCH, 1), jnp.int32),
    )(xn, xn2, centroids, c2)


_NC, _NS, _L = 2, 16, 16
_NSL = 4
_NG = 8
_CS = EMBED_DIM // _NSL
_RG = BATCH // _NG
_CHR = 256
_PAD = _CS + 1

_sc_mesh = plsc.VectorSubcoreMesh(core_axis_name="c", subcore_axis_name="s")


@functools.partial(
    pl.kernel,
    mesh=_sc_mesh,
    out_type=jax.ShapeDtypeStruct((BATCH, EMBED_DIM), jnp.float32),
    scratch_types=[
        pltpu.VMEM((_RG,), jnp.int32),
        pltpu.VMEM((N_CLUSTERS, _PAD), jnp.float32),
        pltpu.VMEM((_CHR, _PAD), jnp.float32),
        pltpu.VMEM((_CHR, _PAD), jnp.float32),
        pltpu.SemaphoreType.DMA,
        pltpu.SemaphoreType.DMA,
    ],
    compiler_params=pltpu.CompilerParams(use_tc_tiling_on_sc=False,
                                        needs_layout_passes=False),
)
def _sc_gather(idx8_hbm, cent_hbm, quant_hbm, idxbuf, tbl, ob0, ob1,
               s0, s1):
    cid = lax.axis_index("c")
    sid = lax.axis_index("s")
    sl = sid % _NSL
    rg = cid * (_NG // _NC) + sid // _NSL
    base = rg * _RG
    col0 = sl * _CS
    pltpu.sync_copy(idx8_hbm.at[rg], idxbuf)
    pltpu.sync_copy(cent_hbm.at[:, pl.ds(col0, _CS)], tbl.at[:, pl.ds(0, _CS)])
    lane = lax.iota(jnp.int32, _L)
    obufs = (ob0, ob1)
    sems = (s0, s1)
    nchunk = _RG // _CHR

    def _chunk(ci, carry):
        for p in range(2):
            @pl.when(lax.rem(ci, 2) == p)
            def _():
                ob = obufs[p]
                sem = sems[p]

                @pl.when(ci >= 2)
                def _():
                    pltpu.make_async_copy(
                        ob.at[:, pl.ds(0, _CS)],
                        quant_hbm.at[pl.ds(base, _CHR), pl.ds(col0, _CS)],
                        sem).wait()

                def _group(gi, c2):
                    idx16 = idxbuf[pl.ds(ci * _CHR + gi * _L, _L)]
                    row16 = gi * _L + lane
                    for c0 in range(0, _CS, 16):
                        ccs = [jnp.full((_L,), c0 + k, jnp.int32)
                               for k in range(16)]
                        vs = [plsc.load_gather(tbl, [idx16, cc])
                              for cc in ccs]
                        for cc, v in zip(ccs, vs):
                            plsc.store_scatter(ob, [row16, cc], v)
                    return c2

                lax.fori_loop(0, _CHR // _L, _group, 0, unroll=2)
                pltpu.async_copy(
                    ob.at[:, pl.ds(0, _CS)],
                    quant_hbm.at[pl.ds(base + ci * _CHR, _CHR),
                                 pl.ds(col0, _CS)], sem)
        return carry

    lax.fori_loop(0, nchunk, _chunk, 0)
    for p in range(2):
        pltpu.make_async_copy(
            obufs[p].at[:, pl.ds(0, _CS)],
            quant_hbm.at[pl.ds(base, _CHR), pl.ds(col0, _CS)],
            sems[p]).wait()


def _update_body(idx_ref, x_ref, ema_sz_ref, ema_w_ref, out_ref,
                 dw_acc, cnt_acc):
    i = pl.program_id(0)
    ii = lax.broadcasted_iota(jnp.int32, (_BB, N_CLUSTERS), 1)
    oh = (ii == idx_ref[...]).astype(jnp.float32)
    ones_col = jnp.ones((_BB, 1), jnp.float32)
    dw_p = lax.dot_general(oh, x_ref[...], (((0,), (0,)), ((), ())))
    cnt_p = lax.dot_general(oh, ones_col, (((0,), (0,)), ((), ())))

    @pl.when(i == 0)
    def _():
        dw_acc[...] = dw_p
        cnt_acc[...] = cnt_p

    @pl.when(i > 0)
    def _():
        dw_acc[...] += dw_p
        cnt_acc[...] += cnt_p

    @pl.when(i == pl.num_programs(0) - 1)
    def _():
        ns = ema_sz_ref[...] * DECAY + (1.0 - DECAY) * cnt_acc[...]
        n = jnp.sum(ns)
        ns2 = (ns + EPS) / (n + N_CLUSTERS * EPS) * n
        new_ema_w = ema_w_ref[...] * DECAY + (1.0 - DECAY) * dw_acc[...]
        out_ref[...] = new_ema_w / ns2


def _update(idx2, x, ema_sz, ema_w):
    return pl.pallas_call(
        _update_body,
        grid=(BATCH // _BB,),
        in_specs=[
            pl.BlockSpec((_BB, 1), lambda i: (i, 0)),
            pl.BlockSpec((_BB, EMBED_DIM), lambda i: (i, 0)),
            pl.BlockSpec((N_CLUSTERS, 1), lambda i: (0, 0)),
            pl.BlockSpec((N_CLUSTERS, EMBED_DIM), lambda i: (0, 0)),
        ],
        out_specs=pl.BlockSpec((N_CLUSTERS, EMBED_DIM), lambda i: (0, 0)),
        out_shape=jax.ShapeDtypeStruct((N_CLUSTERS, EMBED_DIM), jnp.float32),
        scratch_shapes=[
            pltpu.VMEM((N_CLUSTERS, EMBED_DIM), jnp.float32),
            pltpu.VMEM((N_CLUSTERS, 1), jnp.float32),
        ],
    )(idx2, x, ema_sz, ema_w)


def kernel(X, centroids, ema_cluster_size, ema_w):
    norm = jnp.linalg.norm(X, ord=2, axis=1, keepdims=True)
    Xn = X / jnp.clip(norm, 1e-12, None)
    xn2 = jnp.sum(Xn ** 2, axis=1, keepdims=True)
    c2 = jnp.sum(centroids ** 2, axis=1)[None, :]

    idx2 = _assign(Xn, xn2, c2, centroids)
    idx8 = idx2.reshape(_NG, _RG)

    new_centroids = _update(idx2, X, ema_cluster_size.reshape(N_CLUSTERS, 1),
                            ema_w)
    quant = _sc_gather(idx8, centroids)
    return quant, idx2, new_centroids


# speedup vs baseline: 1.1468x; 1.0634x over previous
import functools

import jax
import jax.numpy as jnp
from jax import lax
from jax.experimental import pallas as pl
from jax.experimental.pallas import tpu as pltpu
from jax.experimental.pallas import tpu_sc as plsc

N_CLUSTERS = 1024
EMBED_DIM = 256
DECAY = 0.99
EPS = 1e-05
BATCH = 16384

_BB = 2048


def _assign_body(xn_ref, xn2_ref, c_ref, c2_ref, idx_ref):
    xn = xn_ref[...]
    c = c_ref[...]
    mm = lax.dot_general(xn, c, (((1,), (1,)), ((), ())))
    d = (xn2_ref[...] + c2_ref[...]) - 2.0 * mm
    m = jnp.min(d, axis=1, keepdims=True)
    ii = lax.broadcasted_iota(jnp.int32, d.shape, 1)
    idx_ref[...] = jnp.min(jnp.where(d == m, ii, N_CLUSTERS), axis=1,
                           keepdims=True)


def _assign(xn, xn2, c2, centroids):
    return pl.pallas_call(
        _assign_body,
        grid=(BATCH // _BB,),
        in_specs=[
            pl.BlockSpec((_BB, EMBED_DIM), lambda i: (i, 0)),
            pl.BlockSpec((_BB, 1), lambda i: (i, 0)),
            pl.BlockSpec((N_CLUSTERS, EMBED_DIM), lambda i: (0, 0)),
            pl.BlockSpec((1, N_CLUSTERS), lambda i: (0, 0)),
        ],
        out_specs=pl.BlockSpec((_BB, 1), lambda i: (i, 0)),
        out_shape=jax.ShapeDtypeStruct((BATCH, 1), jnp.int32),
    )(xn, xn2, centroids, c2)


_NC, _NS, _L = 2, 16, 16
_NSL = 4
_NG = 8
_CS = EMBED_DIM // _NSL
_RG = BATCH // _NG
_CHR = 256
_PAD = _CS + 1

_sc_mesh = plsc.VectorSubcoreMesh(core_axis_name="c", subcore_axis_name="s")


@functools.partial(
    pl.kernel,
    mesh=_sc_mesh,
    out_type=jax.ShapeDtypeStruct((BATCH, EMBED_DIM), jnp.float32),
    scratch_types=[
        pltpu.VMEM((_RG,), jnp.int32),
        pltpu.VMEM((N_CLUSTERS, _PAD), jnp.float32),
        pltpu.VMEM((_CHR, _PAD), jnp.float32),
        pltpu.VMEM((_CHR, _PAD), jnp.float32),
        pltpu.SemaphoreType.DMA,
        pltpu.SemaphoreType.DMA,
    ],
    compiler_params=pltpu.CompilerParams(use_tc_tiling_on_sc=False,
                                        needs_layout_passes=False),
)
def _sc_gather(idx8_hbm, cent_hbm, quant_hbm, idxbuf, tbl, ob0, ob1,
               s0, s1):
    cid = lax.axis_index("c")
    sid = lax.axis_index("s")
    sl = sid % _NSL
    rg = cid * (_NG // _NC) + sid // _NSL
    base = rg * _RG
    col0 = sl * _CS
    pltpu.sync_copy(idx8_hbm.at[rg], idxbuf)
    pltpu.sync_copy(cent_hbm.at[:, pl.ds(col0, _CS)], tbl.at[:, pl.ds(0, _CS)])
    lane = lax.iota(jnp.int32, _L)
    obufs = (ob0, ob1)
    sems = (s0, s1)
    nchunk = _RG // _CHR

    def _chunk(ci, carry):
        for p in range(2):
            @pl.when(lax.rem(ci, 2) == p)
            def _():
                ob = obufs[p]
                sem = sems[p]

                @pl.when(ci >= 2)
                def _():
                    pltpu.make_async_copy(
                        ob.at[:, pl.ds(0, _CS)],
                        quant_hbm.at[pl.ds(base, _CHR), pl.ds(col0, _CS)],
                        sem).wait()

                colvecs = [lane + (cb * _L) for cb in range(_CS // _L)]

                def _group(gi, c2):
                    idx16 = idxbuf[pl.ds(ci * _CHR + gi * _L, _L)]
                    for k in range(_L):
                        kk = jnp.full((_L,), k, jnp.int32)
                        rbc = idx16[kk]
                        vs = [plsc.load_gather(tbl, [rbc, cv])
                              for cv in colvecs]
                        for cb, v in enumerate(vs):
                            ob[gi * _L + k, pl.ds(cb * _L, _L)] = v
                    return c2

                lax.fori_loop(0, _CHR // _L, _group, 0)
                pltpu.async_copy(
                    ob.at[:, pl.ds(0, _CS)],
                    quant_hbm.at[pl.ds(base + ci * _CHR, _CHR),
                                 pl.ds(col0, _CS)], sem)
        return carry

    lax.fori_loop(0, nchunk, _chunk, 0)
    for p in range(2):
        pltpu.make_async_copy(
            obufs[p].at[:, pl.ds(0, _CS)],
            quant_hbm.at[pl.ds(base, _CHR), pl.ds(col0, _CS)],
            sems[p]).wait()


def _update_body(idx_ref, x_ref, ema_sz_ref, ema_w_ref, out_ref,
                 dw_acc, cnt_acc):
    i = pl.program_id(0)
    ii = lax.broadcasted_iota(jnp.int32, (_BB, N_CLUSTERS), 1)
    oh = (ii == idx_ref[...]).astype(jnp.float32)
    ones_col = jnp.ones((_BB, 1), jnp.float32)
    dw_p = lax.dot_general(oh, x_ref[...], (((0,), (0,)), ((), ())))
    cnt_p = lax.dot_general(oh, ones_col, (((0,), (0,)), ((), ())))

    @pl.when(i == 0)
    def _():
        dw_acc[...] = dw_p
        cnt_acc[...] = cnt_p

    @pl.when(i > 0)
    def _():
        dw_acc[...] += dw_p
        cnt_acc[...] += cnt_p

    @pl.when(i == pl.num_programs(0) - 1)
    def _():
        ns = ema_sz_ref[...] * DECAY + (1.0 - DECAY) * cnt_acc[...]
        n = jnp.sum(ns)
        ns2 = (ns + EPS) / (n + N_CLUSTERS * EPS) * n
        new_ema_w = ema_w_ref[...] * DECAY + (1.0 - DECAY) * dw_acc[...]
        out_ref[...] = new_ema_w / ns2


def _update(idx2, x, ema_sz, ema_w):
    return pl.pallas_call(
        _update_body,
        grid=(BATCH // _BB,),
        in_specs=[
            pl.BlockSpec((_BB, 1), lambda i: (i, 0)),
            pl.BlockSpec((_BB, EMBED_DIM), lambda i: (i, 0)),
            pl.BlockSpec((N_CLUSTERS, 1), lambda i: (0, 0)),
            pl.BlockSpec((N_CLUSTERS, EMBED_DIM), lambda i: (0, 0)),
        ],
        out_specs=pl.BlockSpec((N_CLUSTERS, EMBED_DIM), lambda i: (0, 0)),
        out_shape=jax.ShapeDtypeStruct((N_CLUSTERS, EMBED_DIM), jnp.float32),
        scratch_shapes=[
            pltpu.VMEM((N_CLUSTERS, EMBED_DIM), jnp.float32),
            pltpu.VMEM((N_CLUSTERS, 1), jnp.float32),
        ],
    )(idx2, x, ema_sz, ema_w)


def kernel(X, centroids, ema_cluster_size, ema_w):
    norm = jnp.linalg.norm(X, ord=2, axis=1, keepdims=True)
    Xn = X / jnp.clip(norm, 1e-12, None)
    xn2 = jnp.sum(Xn ** 2, axis=1, keepdims=True)
    c2 = jnp.sum(centroids ** 2, axis=1)[None, :]

    idx2 = _assign(Xn, xn2, c2, centroids)
    idx8 = idx2.reshape(_NG, _RG)

    new_centroids = _update(idx2, X, ema_cluster_size.reshape(N_CLUSTERS, 1),
                            ema_w)
    quant = _sc_gather(idx8, centroids)
    return quant, idx2, new_centroids


# speedup vs baseline: 1.3229x; 1.1536x over previous
import functools

import jax
import jax.numpy as jnp
from jax import lax
from jax.experimental import pallas as pl
from jax.experimental.pallas import tpu as pltpu
from jax.experimental.pallas import tpu_sc as plsc

N_CLUSTERS = 1024
EMBED_DIM = 256
DECAY = 0.99
EPS = 1e-05
BATCH = 16384

_BB = 2048


def _assign_body(x_ref, c_ref, c2_ref, idx_ref):
    x = x_ref[...]
    c = c_ref[...]
    norm = jnp.sqrt(jnp.sum(x * x, axis=1, keepdims=True))
    xn = x / jnp.maximum(norm, 1e-12)
    xn2 = jnp.sum(xn * xn, axis=1, keepdims=True)
    mm = lax.dot_general(xn, c, (((1,), (1,)), ((), ())))
    d = (xn2 + c2_ref[...]) - 2.0 * mm
    m = jnp.min(d, axis=1, keepdims=True)
    ii = lax.broadcasted_iota(jnp.int32, d.shape, 1)
    idx_ref[...] = jnp.min(jnp.where(d == m, ii, N_CLUSTERS), axis=1,
                           keepdims=True)


def _assign(x, c2, centroids):
    return pl.pallas_call(
        _assign_body,
        grid=(BATCH // _BB,),
        in_specs=[
            pl.BlockSpec((_BB, EMBED_DIM), lambda i: (i, 0)),
            pl.BlockSpec((N_CLUSTERS, EMBED_DIM), lambda i: (0, 0)),
            pl.BlockSpec((1, N_CLUSTERS), lambda i: (0, 0)),
        ],
        out_specs=pl.BlockSpec((_BB, 1), lambda i: (i, 0)),
        out_shape=jax.ShapeDtypeStruct((BATCH, 1), jnp.int32),
    )(x, centroids, c2)


_NC, _NS, _L = 2, 16, 16
_NSL = 4
_NG = 8
_CS = EMBED_DIM // _NSL
_RG = BATCH // _NG
_CHR = 256
_PAD = _CS + 1

_sc_mesh = plsc.VectorSubcoreMesh(core_axis_name="c", subcore_axis_name="s")


@functools.partial(
    pl.kernel,
    mesh=_sc_mesh,
    out_type=jax.ShapeDtypeStruct((BATCH, EMBED_DIM), jnp.float32),
    scratch_types=[
        pltpu.VMEM((_RG,), jnp.int32),
        pltpu.VMEM((N_CLUSTERS, _PAD), jnp.float32),
        pltpu.VMEM((_CHR, _PAD), jnp.float32),
        pltpu.VMEM((_CHR, _PAD), jnp.float32),
        pltpu.SemaphoreType.DMA,
        pltpu.SemaphoreType.DMA,
    ],
    compiler_params=pltpu.CompilerParams(use_tc_tiling_on_sc=False,
                                        needs_layout_passes=False),
)
def _sc_gather(idx8_hbm, cent_hbm, quant_hbm, idxbuf, tbl, ob0, ob1,
               s0, s1):
    cid = lax.axis_index("c")
    sid = lax.axis_index("s")
    sl = sid % _NSL
    rg = cid * (_NG // _NC) + sid // _NSL
    base = rg * _RG
    col0 = sl * _CS
    pltpu.sync_copy(idx8_hbm.at[rg], idxbuf)
    pltpu.sync_copy(cent_hbm.at[:, pl.ds(col0, _CS)], tbl.at[:, pl.ds(0, _CS)])
    lane = lax.iota(jnp.int32, _L)
    obufs = (ob0, ob1)
    sems = (s0, s1)
    nchunk = _RG // _CHR

    def _chunk(ci, carry):
        for p in range(2):
            @pl.when(lax.rem(ci, 2) == p)
            def _():
                ob = obufs[p]
                sem = sems[p]

                @pl.when(ci >= 2)
                def _():
                    pltpu.make_async_copy(
                        ob.at[:, pl.ds(0, _CS)],
                        quant_hbm.at[pl.ds(base, _CHR), pl.ds(col0, _CS)],
                        sem).wait()

                colvecs = [lane + (cb * _L) for cb in range(_CS // _L)]

                def _group(gi, c2):
                    idx16 = idxbuf[pl.ds(ci * _CHR + gi * _L, _L)]
                    for k in range(_L):
                        kk = jnp.full((_L,), k, jnp.int32)
                        rbc = idx16[kk]
                        vs = [plsc.load_gather(tbl, [rbc, cv])
                              for cv in colvecs]
                        for cb, v in enumerate(vs):
                            ob[gi * _L + k, pl.ds(cb * _L, _L)] = v
                    return c2

                lax.fori_loop(0, _CHR // _L, _group, 0)
                pltpu.async_copy(
                    ob.at[:, pl.ds(0, _CS)],
                    quant_hbm.at[pl.ds(base + ci * _CHR, _CHR),
                                 pl.ds(col0, _CS)], sem)
        return carry

    lax.fori_loop(0, nchunk, _chunk, 0)
    for p in range(2):
        pltpu.make_async_copy(
            obufs[p].at[:, pl.ds(0, _CS)],
            quant_hbm.at[pl.ds(base, _CHR), pl.ds(col0, _CS)],
            sems[p]).wait()


def _update_body(idx_ref, x_ref, ema_sz_ref, ema_w_ref, out_ref,
                 dw_acc, cnt_acc):
    i = pl.program_id(0)
    ii = lax.broadcasted_iota(jnp.int32, (_BB, N_CLUSTERS), 1)
    oh = (ii == idx_ref[...]).astype(jnp.float32)
    ones_col = jnp.ones((_BB, 1), jnp.float32)
    dw_p = lax.dot_general(oh, x_ref[...], (((0,), (0,)), ((), ())))
    cnt_p = lax.dot_general(oh, ones_col, (((0,), (0,)), ((), ())))

    @pl.when(i == 0)
    def _():
        dw_acc[...] = dw_p
        cnt_acc[...] = cnt_p

    @pl.when(i > 0)
    def _():
        dw_acc[...] += dw_p
        cnt_acc[...] += cnt_p

    @pl.when(i == pl.num_programs(0) - 1)
    def _():
        ns = ema_sz_ref[...] * DECAY + (1.0 - DECAY) * cnt_acc[...]
        n = jnp.sum(ns)
        ns2 = (ns + EPS) / (n + N_CLUSTERS * EPS) * n
        new_ema_w = ema_w_ref[...] * DECAY + (1.0 - DECAY) * dw_acc[...]
        out_ref[...] = new_ema_w / ns2


def _update(idx2, x, ema_sz, ema_w):
    return pl.pallas_call(
        _update_body,
        grid=(BATCH // _BB,),
        in_specs=[
            pl.BlockSpec((_BB, 1), lambda i: (i, 0)),
            pl.BlockSpec((_BB, EMBED_DIM), lambda i: (i, 0)),
            pl.BlockSpec((N_CLUSTERS, 1), lambda i: (0, 0)),
            pl.BlockSpec((N_CLUSTERS, EMBED_DIM), lambda i: (0, 0)),
        ],
        out_specs=pl.BlockSpec((N_CLUSTERS, EMBED_DIM), lambda i: (0, 0)),
        out_shape=jax.ShapeDtypeStruct((N_CLUSTERS, EMBED_DIM), jnp.float32),
        scratch_shapes=[
            pltpu.VMEM((N_CLUSTERS, EMBED_DIM), jnp.float32),
            pltpu.VMEM((N_CLUSTERS, 1), jnp.float32),
        ],
    )(idx2, x, ema_sz, ema_w)


def kernel(X, centroids, ema_cluster_size, ema_w):
    c2 = jnp.sum(centroids ** 2, axis=1)[None, :]

    idx2 = _assign(X, c2, centroids)
    idx8 = idx2.reshape(_NG, _RG)

    new_centroids = _update(idx2, X, ema_cluster_size.reshape(N_CLUSTERS, 1),
                            ema_w)
    quant = _sc_gather(idx8, centroids)
    return quant, idx2, new_centroids


# speedup vs baseline: 1.3246x; 1.0013x over previous
import functools

import jax
import jax.numpy as jnp
from jax import lax
from jax.experimental import pallas as pl
from jax.experimental.pallas import tpu as pltpu
from jax.experimental.pallas import tpu_sc as plsc

N_CLUSTERS = 1024
EMBED_DIM = 256
DECAY = 0.99
EPS = 1e-05
BATCH = 16384

_BB = 4096


def _assign_body(x_ref, c_ref, c2_ref, idx_ref):
    x = x_ref[...]
    c = c_ref[...]
    norm = jnp.sqrt(jnp.sum(x * x, axis=1, keepdims=True))
    xn = x / jnp.maximum(norm, 1e-12)
    xn2 = jnp.sum(xn * xn, axis=1, keepdims=True)
    mm = lax.dot_general(xn, c, (((1,), (1,)), ((), ())))
    d = (xn2 + c2_ref[...]) - 2.0 * mm
    m = jnp.min(d, axis=1, keepdims=True)
    ii = lax.broadcasted_iota(jnp.int32, d.shape, 1)
    idx_ref[...] = jnp.min(jnp.where(d == m, ii, N_CLUSTERS), axis=1,
                           keepdims=True)


def _assign(x, c2, centroids):
    return pl.pallas_call(
        _assign_body,
        grid=(BATCH // _BB,),
        in_specs=[
            pl.BlockSpec((_BB, EMBED_DIM), lambda i: (i, 0)),
            pl.BlockSpec((N_CLUSTERS, EMBED_DIM), lambda i: (0, 0)),
            pl.BlockSpec((1, N_CLUSTERS), lambda i: (0, 0)),
        ],
        out_specs=pl.BlockSpec((_BB, 1), lambda i: (i, 0)),
        out_shape=jax.ShapeDtypeStruct((BATCH, 1), jnp.int32),
    )(x, centroids, c2)


_NC, _NS, _L = 2, 16, 16
_NSL = 4
_NG = 8
_CS = EMBED_DIM // _NSL
_RG = BATCH // _NG
_CHR = 256
_PAD = _CS + 1

_sc_mesh = plsc.VectorSubcoreMesh(core_axis_name="c", subcore_axis_name="s")


@functools.partial(
    pl.kernel,
    mesh=_sc_mesh,
    out_type=jax.ShapeDtypeStruct((BATCH, EMBED_DIM), jnp.float32),
    scratch_types=[
        pltpu.VMEM((_RG,), jnp.int32),
        pltpu.VMEM((N_CLUSTERS, _PAD), jnp.float32),
        pltpu.VMEM((_CHR, _PAD), jnp.float32),
        pltpu.VMEM((_CHR, _PAD), jnp.float32),
        pltpu.SemaphoreType.DMA,
        pltpu.SemaphoreType.DMA,
    ],
    compiler_params=pltpu.CompilerParams(use_tc_tiling_on_sc=False,
                                        needs_layout_passes=False),
)
def _sc_gather(idx8_hbm, cent_hbm, quant_hbm, idxbuf, tbl, ob0, ob1,
               s0, s1):
    cid = lax.axis_index("c")
    sid = lax.axis_index("s")
    sl = sid % _NSL
    rg = cid * (_NG // _NC) + sid // _NSL
    base = rg * _RG
    col0 = sl * _CS
    pltpu.sync_copy(idx8_hbm.at[rg], idxbuf)
    pltpu.sync_copy(cent_hbm.at[:, pl.ds(col0, _CS)], tbl.at[:, pl.ds(0, _CS)])
    lane = lax.iota(jnp.int32, _L)
    obufs = (ob0, ob1)
    sems = (s0, s1)
    nchunk = _RG // _CHR

    def _chunk(ci, carry):
        for p in range(2):
            @pl.when(lax.rem(ci, 2) == p)
            def _():
                ob = obufs[p]
                sem = sems[p]

                @pl.when(ci >= 2)
                def _():
                    pltpu.make_async_copy(
                        ob.at[:, pl.ds(0, _CS)],
                        quant_hbm.at[pl.ds(base, _CHR), pl.ds(col0, _CS)],
                        sem).wait()

                colvecs = [lane + (cb * _L) for cb in range(_CS // _L)]

                def _group(gi, c2):
                    idx16 = idxbuf[pl.ds(ci * _CHR + gi * _L, _L)]
                    for k in range(_L):
                        kk = jnp.full((_L,), k, jnp.int32)
                        rbc = idx16[kk]
                        vs = [plsc.load_gather(tbl, [rbc, cv])
                              for cv in colvecs]
                        for cb, v in enumerate(vs):
                            ob[gi * _L + k, pl.ds(cb * _L, _L)] = v
                    return c2

                lax.fori_loop(0, _CHR // _L, _group, 0)
                pltpu.async_copy(
                    ob.at[:, pl.ds(0, _CS)],
                    quant_hbm.at[pl.ds(base + ci * _CHR, _CHR),
                                 pl.ds(col0, _CS)], sem)
        return carry

    lax.fori_loop(0, nchunk, _chunk, 0)
    for p in range(2):
        pltpu.make_async_copy(
            obufs[p].at[:, pl.ds(0, _CS)],
            quant_hbm.at[pl.ds(base, _CHR), pl.ds(col0, _CS)],
            sems[p]).wait()


def _update_body(idx_ref, x_ref, ema_sz_ref, ema_w_ref, out_ref,
                 dw_acc, cnt_acc):
    i = pl.program_id(0)
    ii = lax.broadcasted_iota(jnp.int32, (_BB, N_CLUSTERS), 1)
    oh = (ii == idx_ref[...]).astype(jnp.float32)
    ones_col = jnp.ones((_BB, 1), jnp.float32)
    dw_p = lax.dot_general(oh, x_ref[...], (((0,), (0,)), ((), ())))
    cnt_p = lax.dot_general(oh, ones_col, (((0,), (0,)), ((), ())))

    @pl.when(i == 0)
    def _():
        dw_acc[...] = dw_p
        cnt_acc[...] = cnt_p

    @pl.when(i > 0)
    def _():
        dw_acc[...] += dw_p
        cnt_acc[...] += cnt_p

    @pl.when(i == pl.num_programs(0) - 1)
    def _():
        ns = ema_sz_ref[...] * DECAY + (1.0 - DECAY) * cnt_acc[...]
        n = jnp.sum(ns)
        ns2 = (ns + EPS) / (n + N_CLUSTERS * EPS) * n
        new_ema_w = ema_w_ref[...] * DECAY + (1.0 - DECAY) * dw_acc[...]
        out_ref[...] = new_ema_w / ns2


def _update(idx2, x, ema_sz, ema_w):
    return pl.pallas_call(
        _update_body,
        grid=(BATCH // _BB,),
        in_specs=[
            pl.BlockSpec((_BB, 1), lambda i: (i, 0)),
            pl.BlockSpec((_BB, EMBED_DIM), lambda i: (i, 0)),
            pl.BlockSpec((N_CLUSTERS, 1), lambda i: (0, 0)),
            pl.BlockSpec((N_CLUSTERS, EMBED_DIM), lambda i: (0, 0)),
        ],
        out_specs=pl.BlockSpec((N_CLUSTERS, EMBED_DIM), lambda i: (0, 0)),
        out_shape=jax.ShapeDtypeStruct((N_CLUSTERS, EMBED_DIM), jnp.float32),
        scratch_shapes=[
            pltpu.VMEM((N_CLUSTERS, EMBED_DIM), jnp.float32),
            pltpu.VMEM((N_CLUSTERS, 1), jnp.float32),
        ],
    )(idx2, x, ema_sz, ema_w)


def kernel(X, centroids, ema_cluster_size, ema_w):
    c2 = jnp.sum(centroids ** 2, axis=1)[None, :]

    idx2 = _assign(X, c2, centroids)
    idx8 = idx2.reshape(_NG, _RG)

    new_centroids = _update(idx2, X, ema_cluster_size.reshape(N_CLUSTERS, 1),
                            ema_w)
    quant = _sc_gather(idx8, centroids)
    return quant, idx2, new_centroids


# speedup vs baseline: 1.3280x; 1.0026x over previous
import functools

import jax
import jax.numpy as jnp
from jax import lax
from jax.experimental import pallas as pl
from jax.experimental.pallas import tpu as pltpu
from jax.experimental.pallas import tpu_sc as plsc

N_CLUSTERS = 1024
EMBED_DIM = 256
DECAY = 0.99
EPS = 1e-05
BATCH = 16384

_BB = 2048


def _assign_body(x_ref, c_ref, c2_ref, idx_ref):
    x = x_ref[...]
    c = c_ref[...]
    norm = jnp.sqrt(jnp.sum(x * x, axis=1, keepdims=True))
    xn = x / jnp.maximum(norm, 1e-12)
    xn2 = jnp.sum(xn * xn, axis=1, keepdims=True)
    mm = lax.dot_general(xn, c, (((1,), (1,)), ((), ())))
    d = (xn2 + c2_ref[...]) - 2.0 * mm
    m = jnp.min(d, axis=1, keepdims=True)
    ii = lax.broadcasted_iota(jnp.int32, d.shape, 1)
    idx_ref[...] = jnp.min(jnp.where(d == m, ii, N_CLUSTERS), axis=1,
                           keepdims=True)


def _assign(x, c2, centroids):
    return pl.pallas_call(
        _assign_body,
        grid=(BATCH // _BB,),
        in_specs=[
            pl.BlockSpec((_BB, EMBED_DIM), lambda i: (i, 0)),
            pl.BlockSpec((N_CLUSTERS, EMBED_DIM), lambda i: (0, 0)),
            pl.BlockSpec((1, N_CLUSTERS), lambda i: (0, 0)),
        ],
        out_specs=pl.BlockSpec((_BB, 1), lambda i: (i, 0)),
        out_shape=jax.ShapeDtypeStruct((BATCH, 1), jnp.int32),
    )(x, centroids, c2)


_NC, _NS, _L = 2, 16, 16
_NSL = 4
_NG = 8
_CS = EMBED_DIM // _NSL
_RG = BATCH // _NG
_CHR = 256
_PAD = _CS + 1

_sc_mesh = plsc.VectorSubcoreMesh(core_axis_name="c", subcore_axis_name="s")


@functools.partial(
    pl.kernel,
    mesh=_sc_mesh,
    out_type=jax.ShapeDtypeStruct((BATCH, EMBED_DIM), jnp.float32),
    scratch_types=[
        pltpu.VMEM((_RG,), jnp.int32),
        pltpu.VMEM((N_CLUSTERS, _PAD), jnp.float32),
        pltpu.VMEM((_CHR, _PAD), jnp.float32),
        pltpu.VMEM((_CHR, _PAD), jnp.float32),
        pltpu.SemaphoreType.DMA,
        pltpu.SemaphoreType.DMA,
    ],
    compiler_params=pltpu.CompilerParams(use_tc_tiling_on_sc=False,
                                        needs_layout_passes=False),
)
def _sc_gather(idx8_hbm, cent_hbm, quant_hbm, idxbuf, tbl, ob0, ob1,
               s0, s1):
    cid = lax.axis_index("c")
    sid = lax.axis_index("s")
    sl = sid % _NSL
    rg = cid * (_NG // _NC) + sid // _NSL
    base = rg * _RG
    col0 = sl * _CS
    pltpu.sync_copy(idx8_hbm.at[rg], idxbuf)
    pltpu.sync_copy(cent_hbm.at[:, pl.ds(col0, _CS)], tbl.at[:, pl.ds(0, _CS)])
    lane = lax.iota(jnp.int32, _L)
    obufs = (ob0, ob1)
    sems = (s0, s1)
    nchunk = _RG // _CHR

    def _chunk(ci, carry):
        for p in range(2):
            @pl.when(lax.rem(ci, 2) == p)
            def _():
                ob = obufs[p]
                sem = sems[p]

                @pl.when(ci >= 2)
                def _():
                    pltpu.make_async_copy(
                        ob.at[:, pl.ds(0, _CS)],
                        quant_hbm.at[pl.ds(base, _CHR), pl.ds(col0, _CS)],
                        sem).wait()

                colvecs = [lane + (cb * _L) for cb in range(_CS // _L)]

                def _group(gi, c2):
                    idx16 = idxbuf[pl.ds(ci * _CHR + gi * _L, _L)]
                    for k in range(_L):
                        kk = jnp.full((_L,), k, jnp.int32)
                        rbc = idx16[kk]
                        vs = [plsc.load_gather(tbl, [rbc, cv])
                              for cv in colvecs]
                        for cb, v in enumerate(vs):
                            ob[gi * _L + k, pl.ds(cb * _L, _L)] = v
                    return c2

                lax.fori_loop(0, _CHR // _L, _group, 0)
                pltpu.async_copy(
                    ob.at[:, pl.ds(0, _CS)],
                    quant_hbm.at[pl.ds(base + ci * _CHR, _CHR),
                                 pl.ds(col0, _CS)], sem)
        return carry

    lax.fori_loop(0, nchunk, _chunk, 0)
    for p in range(2):
        pltpu.make_async_copy(
            obufs[p].at[:, pl.ds(0, _CS)],
            quant_hbm.at[pl.ds(base, _CHR), pl.ds(col0, _CS)],
            sems[p]).wait()


def _update_body(idx_ref, x_ref, ema_sz_ref, ema_w_ref, out_ref,
                 dw_acc, cnt_acc):
    i = pl.program_id(0)
    ii = lax.broadcasted_iota(jnp.int32, (_BB, N_CLUSTERS), 1)
    oh = (ii == idx_ref[...]).astype(jnp.float32)
    ones_col = jnp.ones((_BB, 1), jnp.float32)
    dw_p = lax.dot_general(oh, x_ref[...], (((0,), (0,)), ((), ())))
    cnt_p = lax.dot_general(oh, ones_col, (((0,), (0,)), ((), ())))

    @pl.when(i == 0)
    def _():
        dw_acc[...] = dw_p
        cnt_acc[...] = cnt_p

    @pl.when(i > 0)
    def _():
        dw_acc[...] += dw_p
        cnt_acc[...] += cnt_p

    @pl.when(i == pl.num_programs(0) - 1)
    def _():
        ns = ema_sz_ref[...] * DECAY + (1.0 - DECAY) * cnt_acc[...]
        n = jnp.sum(ns)
        ns2 = (ns + EPS) / (n + N_CLUSTERS * EPS) * n
        new_ema_w = ema_w_ref[...] * DECAY + (1.0 - DECAY) * dw_acc[...]
        out_ref[...] = new_ema_w / ns2


def _update(idx2, x, ema_sz, ema_w):
    return pl.pallas_call(
        _update_body,
        grid=(BATCH // _BB,),
        in_specs=[
            pl.BlockSpec((_BB, 1), lambda i: (i, 0)),
            pl.BlockSpec((_BB, EMBED_DIM), lambda i: (i, 0)),
            pl.BlockSpec((N_CLUSTERS, 1), lambda i: (0, 0)),
            pl.BlockSpec((N_CLUSTERS, EMBED_DIM), lambda i: (0, 0)),
        ],
        out_specs=pl.BlockSpec((N_CLUSTERS, EMBED_DIM), lambda i: (0, 0)),
        out_shape=jax.ShapeDtypeStruct((N_CLUSTERS, EMBED_DIM), jnp.float32),
        scratch_shapes=[
            pltpu.VMEM((N_CLUSTERS, EMBED_DIM), jnp.float32),
            pltpu.VMEM((N_CLUSTERS, 1), jnp.float32),
        ],
    )(idx2, x, ema_sz, ema_w)


def kernel(X, centroids, ema_cluster_size, ema_w):
    c2 = jnp.sum(centroids ** 2, axis=1)[None, :]

    idx2 = _assign(X, c2, centroids)
    idx8 = idx2.reshape(_NG, _RG)

    new_centroids = _update(idx2, X, ema_cluster_size.reshape(N_CLUSTERS, 1),
                            ema_w)
    quant = _sc_gather(idx8, centroids)
    return quant, idx2, new_centroids
